# baseline (speedup 1.0000x reference)
import functools

import jax
import jax.numpy as jnp
from jax import lax
from jax.experimental import pallas as pl
from jax.experimental.pallas import tpu as pltpu
from jax.experimental.pallas import tpu_sc as plsc

NC = 2
NS = 16
LANES = 16

N = 10000
E = 320000
D = 128
H = 128

EB = 48
NBLK = 210
EPW = NBLK * EB
E2 = NC * NS * EPW
TROWS = 10016
RPT = 624
TAIL = N - NS * RPT


def _interleave_perm():
    perm = []
    for j in range(H // 32):
        for t in range(16):
            perm.extend((32 * j + t, 32 * j + 16 + t))
    return jnp.array(perm, jnp.int32)


def _sc_layer_body(node_hbm, emb_hbm, src_hbm, dst_hbm, out_hbm,
                   idx_s, idx_d, rows_v, emb_v0, emb_v1, emb_v2, msg_v,
                   table_sh, sem_i0, sem_i1, sem_i2, sem_s0, sem_s1, sem_s2):
    c = lax.axis_index("c")
    s = lax.axis_index("s")
    wid = c * NS + s
    in_sems = (sem_i0, sem_i1, sem_i2)
    sc_sems = (sem_s0, sem_s1, sem_s2)
    emb_vs = (emb_v0, emb_v1, emb_v2)

    pltpu.sync_copy(node_hbm.at[pl.ds(s * RPT, RPT)],
                    table_sh.at[pl.ds(s * RPT, RPT)])

    @pl.when(s == 0)
    def _():
        pltpu.sync_copy(node_hbm.at[pl.ds(NS * RPT, TAIL)],
                        table_sh.at[pl.ds(NS * RPT, TAIL)])

    plsc.subcore_barrier()

    def fire_in(g, k):
        base = wid * EPW + g * EB
        pltpu.sync_copy(src_hbm.at[pl.ds(base, EB)], idx_s.at[k])
        pltpu.sync_copy(dst_hbm.at[pl.ds(base, EB)], idx_d.at[k])
        pltpu.async_copy(node_hbm.at[idx_s.at[k]], rows_v.at[k], in_sems[k])
        pltpu.async_copy(emb_hbm.at[wid * NBLK + g], emb_vs[k], in_sems[k])

    def wait_in(k):
        pltpu.make_async_copy(node_hbm.at[idx_s.at[k]], rows_v.at[k],
                              in_sems[k]).wait()
        pltpu.make_async_copy(emb_hbm.at[0], emb_vs[k], in_sems[k]).wait()

    def fire_sc(k):
        pltpu.async_copy(msg_v.at[k], table_sh.at[idx_d.at[k]], sc_sems[k],
                         add=True)

    def wait_sc(k):
        pltpu.make_async_copy(msg_v.at[k], table_sh.at[idx_d.at[k]],
                              sc_sems[k]).wait()

    hi_mask = jnp.full((LANES,), -65536, jnp.int32)

    def mul(k):
        def mul_pair(m, carry):
            for h in range(2):
                i = 2 * m + h
                for j in range(H // 32):
                    rA = rows_v[k, i, pl.ds(j * 32, 16)]
                    rB = rows_v[k, i, pl.ds(j * 32 + 16, 16)]
                    le = emb_vs[k][m, pl.ds(h * 64 + j * 16, 16)]
                    ae = lax.bitcast_convert_type(le << 16, jnp.float32)
                    be = lax.bitcast_convert_type(le & hi_mask, jnp.float32)
                    msg_v[k, i, pl.ds(j * 32, 16)] = rA * ae
                    msg_v[k, i, pl.ds(j * 32 + 16, 16)] = rB * be
            return carry

        lax.fori_loop(0, EB // 2, mul_pair, 0, unroll=False)

    fire_in(0, 0)
    wait_in(0)
    fire_in(1, 1)
    mul(0)
    fire_sc(0)
    wait_in(1)
    fire_in(2, 2)
    mul(1)
    fire_sc(1)
    wait_in(2)
    wait_sc(0)
    fire_in(3, 0)
    mul(2)
    fire_sc(2)

    def body(i, carry):
        g0 = 3 * i
        for k in range(3):
            g = g0 + k
            kn = (k + 1) % 3
            wait_in(k)
            wait_sc(kn)

            @pl.when(g + 1 < NBLK)
            def _():
                fire_in(g + 1, kn)

            mul(k)
            fire_sc(k)
        return carry

    lax.fori_loop(1, NBLK // 3, body, 0, unroll=False)

    wait_sc(1)
    wait_sc(2)
    plsc.subcore_barrier()

    pltpu.sync_copy(table_sh.at[pl.ds(s * RPT, RPT)],
                    out_hbm.at[c, pl.ds(s * RPT, RPT)])

    @pl.when(s == 0)
    def _():
        pltpu.sync_copy(table_sh.at[pl.ds(NS * RPT, TAIL)],
                        out_hbm.at[c, pl.ds(NS * RPT, TAIL)])


@functools.cache
def _get_sc_layer():
    return pl.kernel(
        _sc_layer_body,
        out_type=jax.ShapeDtypeStruct((NC, N, H), jnp.float32),
        mesh=plsc.VectorSubcoreMesh(core_axis_name="c", subcore_axis_name="s",
                                    num_cores=NC, num_subcores=NS),
        scratch_types=[
            pltpu.VMEM((3, EB), jnp.int32),
            pltpu.VMEM((3, EB), jnp.int32),
            pltpu.VMEM((3, EB, H), jnp.float32),
            pltpu.VMEM((EB // 2, H), jnp.int32),
            pltpu.VMEM((EB // 2, H), jnp.int32),
            pltpu.VMEM((EB // 2, H), jnp.int32),
            pltpu.VMEM((3, EB, H), jnp.float32),
            pltpu.VMEM_SHARED((TROWS, H), jnp.float32),
            pltpu.SemaphoreType.DMA,
            pltpu.SemaphoreType.DMA,
            pltpu.SemaphoreType.DMA,
            pltpu.SemaphoreType.DMA,
            pltpu.SemaphoreType.DMA,
            pltpu.SemaphoreType.DMA,
        ],
    )


def _sc_layer(*args):
    return _get_sc_layer()(*args)


def _matmul_bias_body(x_ref, w_ref, b_ref, o_ref):
    o_ref[...] = jnp.dot(x_ref[...], w_ref[...],
                         preferred_element_type=jnp.float32) + b_ref[...]


def _matmul_bias(x, w_t, b, row_blk):
    rows, k = x.shape
    cols = w_t.shape[1]
    return pl.pallas_call(
        _matmul_bias_body,
        grid=(rows // row_blk,),
        in_specs=[
            pl.BlockSpec((row_blk, k), lambda i: (i, 0)),
            pl.BlockSpec((k, cols), lambda i: (0, 0)),
            pl.BlockSpec((1, cols), lambda i: (0, 0)),
        ],
        out_specs=pl.BlockSpec((row_blk, cols), lambda i: (i, 0)),
        out_shape=jax.ShapeDtypeStruct((rows, cols), jnp.float32),
    )(x, w_t, b)


def _embed_bf_body(x_ref, w_ref, b_ref, obf_ref):
    y = jnp.dot(x_ref[...], w_ref[...],
                preferred_element_type=jnp.float32) + b_ref[...]
    obf_ref[...] = y.astype(jnp.bfloat16)


def _embed_bf(x, w_t, b, row_blk):
    rows, k = x.shape
    cols = w_t.shape[1]
    return pl.pallas_call(
        _embed_bf_body,
        grid=(rows // row_blk,),
        in_specs=[
            pl.BlockSpec((row_blk, k), lambda i: (i, 0)),
            pl.BlockSpec((k, cols), lambda i: (0, 0)),
            pl.BlockSpec((1, cols), lambda i: (0, 0)),
        ],
        out_specs=pl.BlockSpec((row_blk, cols), lambda i: (i, 0)),
        out_shape=jax.ShapeDtypeStruct((rows, cols), jnp.bfloat16),
    )(x, w_t, b)


def _combine_body(a_ref, b_ref, n_ref, o_ref):
    o_ref[...] = a_ref[0] + b_ref[0] - n_ref[...]


def _combine(parts, node):
    row_blk = 2000
    return pl.pallas_call(
        _combine_body,
        grid=(N // row_blk,),
        in_specs=[
            pl.BlockSpec((1, row_blk, H), lambda i: (0, i, 0)),
            pl.BlockSpec((1, row_blk, H), lambda i: (1, i, 0)),
            pl.BlockSpec((row_blk, H), lambda i: (i, 0)),
        ],
        out_specs=pl.BlockSpec((row_blk, H), lambda i: (i, 0)),
        out_shape=jax.ShapeDtypeStruct((N, H), jnp.float32),
    )(parts, parts, node)


NUM_GRAPHS_OUT = 64
MLP_BLK = 2000


def _mlp_pool_body(pa_ref, pb_ref, n_ref, w1_ref, b1_ref, w2_ref, b2_ref,
                   w3_ref, batch_ref, o_ref):
    i = pl.program_id(0)
    h = pa_ref[0] + pb_ref[0] - n_ref[...]
    h = jnp.maximum(h, 0.0)
    h = jnp.dot(h, w1_ref[...], preferred_element_type=jnp.float32) + b1_ref[...]
    h = jnp.maximum(h, 0.0)
    h = jnp.dot(h, w2_ref[...], preferred_element_type=jnp.float32) + b2_ref[...]
    h = jnp.maximum(h, 0.0)
    e = jnp.dot(h, w3_ref[...], preferred_element_type=jnp.float32)
    b = batch_ref[...].reshape(MLP_BLK)
    ids = lax.broadcasted_iota(jnp.int32, (MLP_BLK, NUM_GRAPHS_OUT), 1)
    oh = (b[:, None] == ids).astype(jnp.float32)
    dgp = lax.dot_general(oh, e, (((0,), (0,)), ((), ())),
                          preferred_element_type=jnp.float32)

    @pl.when(i == 0)
    def _():
        o_ref[...] = jnp.zeros_like(o_ref)

    o_ref[...] += dgp


def _mlp_pool(parts, node, w1_t, b1, w2_t, b2, w3_t, batch3):
    grid = N // MLP_BLK
    return pl.pallas_call(
        _mlp_pool_body,
        grid=(grid,),
        in_specs=[
            pl.BlockSpec((1, MLP_BLK, H), lambda i: (0, i, 0)),
            pl.BlockSpec((1, MLP_BLK, H), lambda i: (1, i, 0)),
            pl.BlockSpec((MLP_BLK, H), lambda i: (i, 0)),
            pl.BlockSpec((H, H), lambda i: (0, 0)),
            pl.BlockSpec((1, H), lambda i: (0, 0)),
            pl.BlockSpec((H, H // 2), lambda i: (0, 0)),
            pl.BlockSpec((1, H // 2), lambda i: (0, 0)),
            pl.BlockSpec((H // 2, 1), lambda i: (0, 0)),
            pl.BlockSpec((1, 1, MLP_BLK), lambda i: (i, 0, 0)),
        ],
        out_specs=pl.BlockSpec((NUM_GRAPHS_OUT, 1), lambda i: (0, 0)),
        out_shape=jax.ShapeDtypeStruct((NUM_GRAPHS_OUT, 1), jnp.float32),
    )(parts, parts, node, w1_t, b1, w2_t, b2, w3_t, batch3)


def kernel(x, edge_index, edge_attr, batch, W_atom, b_atom, W_bond, b_bond,
           W1, b1, W2, b2, W3):
    pad = E2 - E
    src = jnp.concatenate([edge_index[0].astype(jnp.int32),
                           jnp.zeros((pad,), jnp.int32)])
    dst = jnp.concatenate([edge_index[1].astype(jnp.int32),
                           jnp.full((pad,), N + 8, jnp.int32)])
    ea_p = jnp.concatenate([edge_attr,
                            jnp.zeros((pad, edge_attr.shape[1]), jnp.float32)])
    batch3 = batch.astype(jnp.int32).reshape(N // MLP_BLK, 1, MLP_BLK)

    perm = _interleave_perm()
    node_emb = _matmul_bias(x, W_atom.T, b_atom.reshape(1, H), 2000)
    emb_bf = _embed_bf(ea_p, W_bond.T[:, perm], b_bond[perm].reshape(1, H),
                       4032)
    emb_i = lax.bitcast_convert_type(emb_bf.reshape(E2, H // 2, 2),
                                     jnp.int32).reshape(NC * NS * NBLK,
                                                        EB // 2, H)

    parts1 = _sc_layer(node_emb, emb_i, src, dst)
    node1 = _combine(parts1, node_emb)
    parts2 = _sc_layer(node1, emb_i, src, dst)

    dg = _mlp_pool(parts2, node1, W1.T, b1.reshape(1, H),
                   W2.T, b2.reshape(1, H // 2), W3.T, batch3)
    return dg


# speedup vs baseline: 1.6498x; 1.6498x over previous
import functools

import jax
import jax.numpy as jnp
from jax import lax
from jax.experimental import pallas as pl
from jax.experimental.pallas import tpu as pltpu
from jax.experimental.pallas import tpu_sc as plsc

NC = 2
NS = 16
LANES = 16

N = 10000
E = 320000
D = 128
H = 128

EB = 48
NBLK = 210
EPW = NBLK * EB
E2 = NC * NS * EPW
TROWS = 10016
RPT = 624
TAIL = N - NS * RPT


def _sc_layer_body(node_hbm, emb_hbm, src_hbm, dst_hbm, out_hbm,
                   idx_s, idx_d, rows_v, emb_v0, emb_v1, emb_v2, msg_v,
                   table_sh, sem_i0, sem_i1, sem_i2, sem_s0, sem_s1, sem_s2):
    c = lax.axis_index("c")
    s = lax.axis_index("s")
    wid = c * NS + s
    in_sems = (sem_i0, sem_i1, sem_i2)
    sc_sems = (sem_s0, sem_s1, sem_s2)
    emb_vs = (emb_v0, emb_v1, emb_v2)

    pltpu.sync_copy(node_hbm.at[pl.ds(s * RPT, RPT)],
                    table_sh.at[pl.ds(s * RPT, RPT)])

    @pl.when(s == 0)
    def _():
        pltpu.sync_copy(node_hbm.at[pl.ds(NS * RPT, TAIL)],
                        table_sh.at[pl.ds(NS * RPT, TAIL)])

    plsc.subcore_barrier()

    def fire_in(g, k):
        base = wid * EPW + g * EB
        pltpu.sync_copy(src_hbm.at[pl.ds(base, EB)], idx_s.at[k])
        pltpu.sync_copy(dst_hbm.at[pl.ds(base, EB)], idx_d.at[k])
        pltpu.async_copy(node_hbm.at[idx_s.at[k]], rows_v.at[k], in_sems[k])
        pltpu.async_copy(emb_hbm.at[pl.ds((wid * NBLK + g) * (EB // 2),
                                          EB // 2)],
                         emb_vs[k], in_sems[k])

    def wait_in(k):
        pltpu.make_async_copy(node_hbm.at[idx_s.at[k]], rows_v.at[k],
                              in_sems[k]).wait()
        pltpu.make_async_copy(emb_hbm.at[pl.ds(0, EB // 2)], emb_vs[k],
                              in_sems[k]).wait()

    def fire_sc(k):
        pltpu.async_copy(msg_v.at[k], table_sh.at[idx_d.at[k]], sc_sems[k],
                         add=True)

    def wait_sc(k):
        pltpu.make_async_copy(msg_v.at[k], table_sh.at[idx_d.at[k]],
                              sc_sems[k]).wait()

    hi_mask = jnp.full((LANES,), -65536, jnp.int32)

    def mul(k):
        def mul_pair(m, carry):
            for h in range(2):
                i = 2 * m + h
                for j in range(H // 32):
                    rA = rows_v[k, i, pl.ds(j * 32, 16)]
                    rB = rows_v[k, i, pl.ds(j * 32 + 16, 16)]
                    le = emb_vs[k][m, pl.ds(h * 64 + j * 16, 16)]
                    ae = lax.bitcast_convert_type(le << 16, jnp.float32)
                    be = lax.bitcast_convert_type(le & hi_mask, jnp.float32)
                    msg_v[k, i, pl.ds(j * 32, 16)] = rA * ae
                    msg_v[k, i, pl.ds(j * 32 + 16, 16)] = rB * be
            return carry

        lax.fori_loop(0, EB // 2, mul_pair, 0, unroll=False)

    fire_in(0, 0)
    wait_in(0)
    fire_in(1, 1)
    mul(0)
    fire_sc(0)
    wait_in(1)
    fire_in(2, 2)
    mul(1)
    fire_sc(1)
    wait_in(2)
    wait_sc(0)
    fire_in(3, 0)
    mul(2)
    fire_sc(2)

    def body(i, carry):
        g0 = 3 * i
        for k in range(3):
            g = g0 + k
            kn = (k + 1) % 3
            wait_in(k)
            wait_sc(kn)

            @pl.when(g + 1 < NBLK)
            def _():
                fire_in(g + 1, kn)

            mul(k)
            fire_sc(k)
        return carry

    lax.fori_loop(1, NBLK // 3, body, 0, unroll=False)

    wait_sc(1)
    wait_sc(2)
    plsc.subcore_barrier()

    pltpu.sync_copy(table_sh.at[pl.ds(s * RPT, RPT)],
                    out_hbm.at[c, pl.ds(s * RPT, RPT)])

    @pl.when(s == 0)
    def _():
        pltpu.sync_copy(table_sh.at[pl.ds(NS * RPT, TAIL)],
                        out_hbm.at[c, pl.ds(NS * RPT, TAIL)])


@functools.cache
def _get_sc_layer():
    return pl.kernel(
        _sc_layer_body,
        out_type=jax.ShapeDtypeStruct((NC, N, H), jnp.float32),
        mesh=plsc.VectorSubcoreMesh(core_axis_name="c", subcore_axis_name="s",
                                    num_cores=NC, num_subcores=NS),
        scratch_types=[
            pltpu.VMEM((3, EB), jnp.int32),
            pltpu.VMEM((3, EB), jnp.int32),
            pltpu.VMEM((3, EB, H), jnp.float32),
            pltpu.VMEM((EB // 2, H), jnp.int32),
            pltpu.VMEM((EB // 2, H), jnp.int32),
            pltpu.VMEM((EB // 2, H), jnp.int32),
            pltpu.VMEM((3, EB, H), jnp.float32),
            pltpu.VMEM_SHARED((TROWS, H), jnp.float32),
            pltpu.SemaphoreType.DMA,
            pltpu.SemaphoreType.DMA,
            pltpu.SemaphoreType.DMA,
            pltpu.SemaphoreType.DMA,
            pltpu.SemaphoreType.DMA,
            pltpu.SemaphoreType.DMA,
        ],
    )


def _sc_layer(*args):
    return _get_sc_layer()(*args)


def _matmul_bias_body(x_ref, w_ref, b_ref, o_ref):
    o_ref[...] = jnp.dot(x_ref[...], w_ref[...],
                         preferred_element_type=jnp.float32) + b_ref[...]


def _matmul_bias(x, w_t, b, row_blk):
    rows, k = x.shape
    cols = w_t.shape[1]
    return pl.pallas_call(
        _matmul_bias_body,
        grid=(rows // row_blk,),
        in_specs=[
            pl.BlockSpec((row_blk, k), lambda i: (i, 0)),
            pl.BlockSpec((k, cols), lambda i: (0, 0)),
            pl.BlockSpec((1, cols), lambda i: (0, 0)),
        ],
        out_specs=pl.BlockSpec((row_blk, cols), lambda i: (i, 0)),
        out_shape=jax.ShapeDtypeStruct((rows, cols), jnp.float32),
    )(x, w_t, b)


def _embed_pack_body(x_ref, wa_ref, wb_ref, ba_ref, bb_ref, o_ref):
    ya = jnp.dot(x_ref[...], wa_ref[...],
                 preferred_element_type=jnp.float32) + ba_ref[...]
    yb = jnp.dot(x_ref[...], wb_ref[...],
                 preferred_element_type=jnp.float32) + bb_ref[...]
    ia = lax.bitcast_convert_type(ya, jnp.int32)
    ib = lax.bitcast_convert_type(yb, jnp.int32)
    ra = ((ia + 0x7FFF + ((ia >> 16) & 1)) >> 16) & 0xFFFF
    rb = ((ib + 0x7FFF + ((ib >> 16) & 1)) >> 16) & 0xFFFF
    o_ref[...] = ra | (rb << 16)


def _embed_pack(x2, wa2, wb2, ba2, bb2, row_blk):
    rows, k = x2.shape
    return pl.pallas_call(
        _embed_pack_body,
        grid=(rows // row_blk,),
        in_specs=[
            pl.BlockSpec((row_blk, k), lambda i: (i, 0)),
            pl.BlockSpec((k, H), lambda i: (0, 0)),
            pl.BlockSpec((k, H), lambda i: (0, 0)),
            pl.BlockSpec((1, H), lambda i: (0, 0)),
            pl.BlockSpec((1, H), lambda i: (0, 0)),
        ],
        out_specs=pl.BlockSpec((row_blk, H), lambda i: (i, 0)),
        out_shape=jax.ShapeDtypeStruct((rows, H), jnp.int32),
    )(x2, wa2, wb2, ba2, bb2)


def _combine_body(a_ref, b_ref, n_ref, o_ref):
    o_ref[...] = a_ref[0] + b_ref[0] - n_ref[...]


def _combine(parts, node):
    row_blk = 2000
    return pl.pallas_call(
        _combine_body,
        grid=(N // row_blk,),
        in_specs=[
            pl.BlockSpec((1, row_blk, H), lambda i: (0, i, 0)),
            pl.BlockSpec((1, row_blk, H), lambda i: (1, i, 0)),
            pl.BlockSpec((row_blk, H), lambda i: (i, 0)),
        ],
        out_specs=pl.BlockSpec((row_blk, H), lambda i: (i, 0)),
        out_shape=jax.ShapeDtypeStruct((N, H), jnp.float32),
    )(parts, parts, node)


NUM_GRAPHS_OUT = 64
MLP_BLK = 2000


def _mlp_pool_body(pa_ref, pb_ref, n_ref, w1_ref, b1_ref, w2_ref, b2_ref,
                   w3_ref, batch_ref, o_ref):
    i = pl.program_id(0)
    h = pa_ref[0] + pb_ref[0] - n_ref[...]
    h = jnp.maximum(h, 0.0)
    h = jnp.dot(h, w1_ref[...], preferred_element_type=jnp.float32) + b1_ref[...]
    h = jnp.maximum(h, 0.0)
    h = jnp.dot(h, w2_ref[...], preferred_element_type=jnp.float32) + b2_ref[...]
    h = jnp.maximum(h, 0.0)
    e = jnp.dot(h, w3_ref[...], preferred_element_type=jnp.float32)
    b = batch_ref[...].reshape(MLP_BLK)
    ids = lax.broadcasted_iota(jnp.int32, (MLP_BLK, NUM_GRAPHS_OUT), 1)
    oh = (b[:, None] == ids).astype(jnp.float32)
    dgp = lax.dot_general(oh, e, (((0,), (0,)), ((), ())),
                          preferred_element_type=jnp.float32)

    @pl.when(i == 0)
    def _():
        o_ref[...] = jnp.zeros_like(o_ref)

    o_ref[...] += dgp


def _mlp_pool(parts, node, w1_t, b1, w2_t, b2, w3_t, batch3):
    grid = N // MLP_BLK
    return pl.pallas_call(
        _mlp_pool_body,
        grid=(grid,),
        in_specs=[
            pl.BlockSpec((1, MLP_BLK, H), lambda i: (0, i, 0)),
            pl.BlockSpec((1, MLP_BLK, H), lambda i: (1, i, 0)),
            pl.BlockSpec((MLP_BLK, H), lambda i: (i, 0)),
            pl.BlockSpec((H, H), lambda i: (0, 0)),
            pl.BlockSpec((1, H), lambda i: (0, 0)),
            pl.BlockSpec((H, H // 2), lambda i: (0, 0)),
            pl.BlockSpec((1, H // 2), lambda i: (0, 0)),
            pl.BlockSpec((H // 2, 1), lambda i: (0, 0)),
            pl.BlockSpec((1, 1, MLP_BLK), lambda i: (i, 0, 0)),
        ],
        out_specs=pl.BlockSpec((NUM_GRAPHS_OUT, 1), lambda i: (0, 0)),
        out_shape=jax.ShapeDtypeStruct((NUM_GRAPHS_OUT, 1), jnp.float32),
    )(parts, parts, node, w1_t, b1, w2_t, b2, w3_t, batch3)


def kernel(x, edge_index, edge_attr, batch, W_atom, b_atom, W_bond, b_bond,
           W1, b1, W2, b2, W3):
    pad = E2 - E
    src = jnp.concatenate([edge_index[0].astype(jnp.int32),
                           jnp.zeros((pad,), jnp.int32)])
    dst = jnp.concatenate([edge_index[1].astype(jnp.int32),
                           jnp.full((pad,), N + 8, jnp.int32)])
    ea_p = jnp.concatenate([edge_attr,
                            jnp.zeros((pad, edge_attr.shape[1]), jnp.float32)])
    batch3 = batch.astype(jnp.int32).reshape(N // MLP_BLK, 1, MLP_BLK)

    perm_a = jnp.arange(H // 2, dtype=jnp.int32)
    perm_a = 32 * (perm_a // 16) + perm_a % 16
    perm_b = perm_a + 16
    wa = W_bond.T[:, perm_a]
    wb = W_bond.T[:, perm_b]
    z = jnp.zeros_like(wa)
    wa2 = jnp.block([[wa, z], [z, wa]])
    wb2 = jnp.block([[wb, z], [z, wb]])
    ba2 = jnp.concatenate([b_bond[perm_a], b_bond[perm_a]]).reshape(1, H)
    bb2 = jnp.concatenate([b_bond[perm_b], b_bond[perm_b]]).reshape(1, H)

    node_emb = _matmul_bias(x, W_atom.T, b_atom.reshape(1, H), 2000)
    emb_i = _embed_pack(ea_p.reshape(E2 // 2, 32), wa2, wb2, ba2, bb2,
                        2016)
    parts1 = _sc_layer(node_emb, emb_i, src, dst)
    node1 = _combine(parts1, node_emb)
    parts2 = _sc_layer(node1, emb_i, src, dst)

    dg = _mlp_pool(parts2, node1, W1.T, b1.reshape(1, H),
                   W2.T, b2.reshape(1, H // 2), W3.T, batch3)
    return dg


# speedup vs baseline: 1.8389x; 1.1146x over previous
import functools

import jax
import jax.numpy as jnp
from jax import lax
from jax.experimental import pallas as pl
from jax.experimental.pallas import tpu as pltpu
from jax.experimental.pallas import tpu_sc as plsc

NC = 2
NS = 16
LANES = 16

N = 10000
E = 320000
D = 128
H = 128

EB = 112
NBLK = 90
EPW = NBLK * EB
E2 = NC * NS * EPW
TROWS = 10016
RPT = 624
TAIL = N - NS * RPT


def _sc_layer_body(node_hbm, emb_hbm, src_hbm, dst_hbm, out_hbm,
                   idx_s, idx_d, rows_v, emb_v, table_sh, sem):
    c = lax.axis_index("c")
    s = lax.axis_index("s")
    wid = c * NS + s

    pltpu.sync_copy(node_hbm.at[pl.ds(s * RPT, RPT)],
                    table_sh.at[pl.ds(s * RPT, RPT)])

    @pl.when(s == 0)
    def _():
        pltpu.sync_copy(node_hbm.at[pl.ds(NS * RPT, TAIL)],
                        table_sh.at[pl.ds(NS * RPT, TAIL)])

    plsc.subcore_barrier()

    def edge_block(g, carry):
        base = wid * EPW + g * EB
        pltpu.sync_copy(src_hbm.at[pl.ds(base, EB)], idx_s)
        pltpu.sync_copy(dst_hbm.at[pl.ds(base, EB)], idx_d)
        gather = pltpu.async_copy(node_hbm.at[idx_s], rows_v, sem)
        pltpu.sync_copy(emb_hbm.at[pl.ds(base, EB)], emb_v)
        gather.wait()

        def mul_row(i, carry2):
            for d in range(H // LANES):
                sl = pl.ds(d * LANES, LANES)
                rows_v[i, sl] = rows_v[i, sl] * emb_v[i, sl]
            return carry2

        lax.fori_loop(0, EB, mul_row, 0, unroll=False)
        pltpu.sync_copy(rows_v, table_sh.at[idx_d], add=True)
        return carry

    lax.fori_loop(0, NBLK, edge_block, 0, unroll=False)
    plsc.subcore_barrier()

    pltpu.sync_copy(table_sh.at[pl.ds(s * RPT, RPT)],
                    out_hbm.at[c, pl.ds(s * RPT, RPT)])

    @pl.when(s == 0)
    def _():
        pltpu.sync_copy(table_sh.at[pl.ds(NS * RPT, TAIL)],
                        out_hbm.at[c, pl.ds(NS * RPT, TAIL)])


@functools.cache
def _get_sc_layer():
    return pl.kernel(
        _sc_layer_body,
        out_type=jax.ShapeDtypeStruct((NC, N, H), jnp.float32),
        mesh=plsc.VectorSubcoreMesh(core_axis_name="c", subcore_axis_name="s",
                                    num_cores=NC, num_subcores=NS),
        scratch_types=[
            pltpu.VMEM((EB,), jnp.int32),
            pltpu.VMEM((EB,), jnp.int32),
            pltpu.VMEM((EB, H), jnp.float32),
            pltpu.VMEM((EB, H), jnp.float32),
            pltpu.VMEM_SHARED((N, H), jnp.float32),
            pltpu.SemaphoreType.DMA,
        ],
    )


def _sc_layer(*args):
    return _get_sc_layer()(*args)


def _matmul_bias_body(x_ref, w_ref, b_ref, o_ref):
    o_ref[...] = jnp.dot(x_ref[...], w_ref[...],
                         preferred_element_type=jnp.float32) + b_ref[...]


def _matmul_bias(x, w_t, b, row_blk):
    rows, k = x.shape
    cols = w_t.shape[1]
    grid = rows // row_blk
    return pl.pallas_call(
        _matmul_bias_body,
        grid=(grid,),
        in_specs=[
            pl.BlockSpec((row_blk, k), lambda i: (i, 0)),
            pl.BlockSpec((k, cols), lambda i: (0, 0)),
            pl.BlockSpec((1, cols), lambda i: (0, 0)),
        ],
        out_specs=pl.BlockSpec((row_blk, cols), lambda i: (i, 0)),
        out_shape=jax.ShapeDtypeStruct((rows, cols), jnp.float32),
    )(x, w_t, b)


def _combine_body(a_ref, b_ref, n_ref, o_ref):
    o_ref[...] = a_ref[0] + b_ref[0] - n_ref[...]


def _combine(parts, node):
    row_blk = 2000
    return pl.pallas_call(
        _combine_body,
        grid=(N // row_blk,),
        in_specs=[
            pl.BlockSpec((1, row_blk, H), lambda i: (0, i, 0)),
            pl.BlockSpec((1, row_blk, H), lambda i: (1, i, 0)),
            pl.BlockSpec((row_blk, H), lambda i: (i, 0)),
        ],
        out_specs=pl.BlockSpec((row_blk, H), lambda i: (i, 0)),
        out_shape=jax.ShapeDtypeStruct((N, H), jnp.float32),
    )(parts, parts, node)


NUM_GRAPHS_OUT = 64
MLP_BLK = 2000


def _mlp_pool_body(pa_ref, pb_ref, n_ref, w1_ref, b1_ref, w2_ref, b2_ref,
                   w3_ref, batch_ref, o_ref):
    i = pl.program_id(0)
    h = pa_ref[0] + pb_ref[0] - n_ref[...]
    h = jnp.maximum(h, 0.0)
    h = jnp.dot(h, w1_ref[...], preferred_element_type=jnp.float32) + b1_ref[...]
    h = jnp.maximum(h, 0.0)
    h = jnp.dot(h, w2_ref[...], preferred_element_type=jnp.float32) + b2_ref[...]
    h = jnp.maximum(h, 0.0)
    e = jnp.dot(h, w3_ref[...], preferred_element_type=jnp.float32)
    b = batch_ref[...].reshape(MLP_BLK)
    ids = lax.broadcasted_iota(jnp.int32, (MLP_BLK, NUM_GRAPHS_OUT), 1)
    oh = (b[:, None] == ids).astype(jnp.float32)
    dgp = lax.dot_general(oh, e, (((0,), (0,)), ((), ())),
                          preferred_element_type=jnp.float32)

    @pl.when(i == 0)
    def _():
        o_ref[...] = jnp.zeros_like(o_ref)

    o_ref[...] += dgp


def _mlp_pool(parts, node, w1_t, b1, w2_t, b2, w3_t, batch3):
    grid = N // MLP_BLK
    return pl.pallas_call(
        _mlp_pool_body,
        grid=(grid,),
        in_specs=[
            pl.BlockSpec((1, MLP_BLK, H), lambda i: (0, i, 0)),
            pl.BlockSpec((1, MLP_BLK, H), lambda i: (1, i, 0)),
            pl.BlockSpec((MLP_BLK, H), lambda i: (i, 0)),
            pl.BlockSpec((H, H), lambda i: (0, 0)),
            pl.BlockSpec((1, H), lambda i: (0, 0)),
            pl.BlockSpec((H, H // 2), lambda i: (0, 0)),
            pl.BlockSpec((1, H // 2), lambda i: (0, 0)),
            pl.BlockSpec((H // 2, 1), lambda i: (0, 0)),
            pl.BlockSpec((1, 1, MLP_BLK), lambda i: (i, 0, 0)),
        ],
        out_specs=pl.BlockSpec((NUM_GRAPHS_OUT, 1), lambda i: (0, 0)),
        out_shape=jax.ShapeDtypeStruct((NUM_GRAPHS_OUT, 1), jnp.float32),
    )(parts, parts, node, w1_t, b1, w2_t, b2, w3_t, batch3)


def kernel(x, edge_index, edge_attr, batch, W_atom, b_atom, W_bond, b_bond,
           W1, b1, W2, b2, W3):
    pad = E2 - E
    src = jnp.concatenate([edge_index[0].astype(jnp.int32),
                           jnp.zeros((pad,), jnp.int32)])
    dst = jnp.concatenate([edge_index[1].astype(jnp.int32),
                           jnp.full((pad,), N + 8, jnp.int32)])
    ea_p = jnp.concatenate([edge_attr,
                            jnp.zeros((pad, edge_attr.shape[1]), jnp.float32)])
    batch3 = batch.astype(jnp.int32).reshape(N // MLP_BLK, 1, MLP_BLK)

    node_emb = _matmul_bias(x, W_atom.T, b_atom.reshape(1, H), 2000)
    edge_emb = _matmul_bias(ea_p, W_bond.T, b_bond.reshape(1, H), 4032)

    parts1 = _sc_layer(node_emb, edge_emb, src, dst)
    node1 = _combine(parts1, node_emb)
    parts2 = _sc_layer(node1, edge_emb, src, dst)

    dg = _mlp_pool(parts2, node1, W1.T, b1.reshape(1, H),
                   W2.T, b2.reshape(1, H // 2), W3.T, batch3)
    return dg


# speedup vs baseline: 2.2155x; 1.2048x over previous
import functools

import jax
import jax.numpy as jnp
from jax import lax
from jax.experimental import pallas as pl
from jax.experimental.pallas import tpu as pltpu
from jax.experimental.pallas import tpu_sc as plsc

NC = 2
NS = 16
LANES = 16

N = 10000
E = 320000
D = 128
H = 128

EPW = E // (NC * NS)
EB = 80
NBLK = EPW // EB
RPT = 624
TAIL = N - NS * RPT


def _sc_layer_body(node_hbm, emb_hbm, src_hbm, dst_hbm, out_hbm,
                   idx_s, idx_d, rows_v, emb_v, table_sh, sem,
                   sc_sem0, sc_sem1):
    c = lax.axis_index("c")
    s = lax.axis_index("s")
    wid = c * NS + s

    pltpu.sync_copy(node_hbm.at[pl.ds(s * RPT, RPT)],
                    table_sh.at[pl.ds(s * RPT, RPT)])

    @pl.when(s == 0)
    def _():
        pltpu.sync_copy(node_hbm.at[pl.ds(NS * RPT, TAIL)],
                        table_sh.at[pl.ds(NS * RPT, TAIL)])

    plsc.subcore_barrier()

    sc_sems = (sc_sem0, sc_sem1)

    def edge_block_slot(g, b):
        base = wid * EPW + g * EB
        pltpu.sync_copy(src_hbm.at[pl.ds(base, EB)], idx_s.at[b])
        pltpu.sync_copy(dst_hbm.at[pl.ds(base, EB)], idx_d.at[b])

        def _wait_prev():
            pltpu.make_async_copy(rows_v.at[b], table_sh.at[idx_d.at[b]],
                                  sc_sems[b]).wait()

        if isinstance(g, int):
            if g >= 2:
                _wait_prev()
        else:
            pl.when(g >= 2)(_wait_prev)

        gather = pltpu.async_copy(node_hbm.at[idx_s.at[b]], rows_v.at[b], sem)
        pltpu.sync_copy(emb_hbm.at[pl.ds(base, EB)], emb_v)
        gather.wait()

        def mul_row(i, carry2):
            for d in range(H // LANES):
                sl = pl.ds(d * LANES, LANES)
                rows_v[b, i, sl] = rows_v[b, i, sl] * emb_v[i, sl]
            return carry2

        lax.fori_loop(0, EB, mul_row, 0, unroll=False)
        pltpu.async_copy(rows_v.at[b], table_sh.at[idx_d.at[b]], sc_sems[b],
                         add=True)

    def edge_block(h, carry):
        edge_block_slot(2 * h, 0)
        edge_block_slot(2 * h + 1, 1)
        return carry

    lax.fori_loop(0, NBLK // 2, edge_block, 0, unroll=False)
    if NBLK % 2:
        edge_block_slot(NBLK - 1, 0)
    pltpu.make_async_copy(rows_v.at[0], table_sh.at[idx_d.at[0]],
                          sc_sems[0]).wait()
    pltpu.make_async_copy(rows_v.at[1], table_sh.at[idx_d.at[1]],
                          sc_sems[1]).wait()
    plsc.subcore_barrier()

    pltpu.sync_copy(table_sh.at[pl.ds(s * RPT, RPT)],
                    out_hbm.at[c, pl.ds(s * RPT, RPT)])

    @pl.when(s == 0)
    def _():
        pltpu.sync_copy(table_sh.at[pl.ds(NS * RPT, TAIL)],
                        out_hbm.at[c, pl.ds(NS * RPT, TAIL)])


@functools.cache
def _get_sc_layer():
    return pl.kernel(
        _sc_layer_body,
        out_type=jax.ShapeDtypeStruct((NC, N, H), jnp.float32),
        mesh=plsc.VectorSubcoreMesh(core_axis_name="c", subcore_axis_name="s",
                                    num_cores=NC, num_subcores=NS),
        scratch_types=[
            pltpu.VMEM((2, EB), jnp.int32),
            pltpu.VMEM((2, EB), jnp.int32),
            pltpu.VMEM((2, EB, H), jnp.float32),
            pltpu.VMEM((EB, H), jnp.float32),
            pltpu.VMEM_SHARED((N, H), jnp.float32),
            pltpu.SemaphoreType.DMA,
            pltpu.SemaphoreType.DMA,
            pltpu.SemaphoreType.DMA,
        ],
    )


def _sc_layer(*args):
    return _get_sc_layer()(*args)


def _matmul_bias_body(x_ref, w_ref, b_ref, o_ref):
    o_ref[...] = jnp.dot(x_ref[...], w_ref[...],
                         preferred_element_type=jnp.float32) + b_ref[...]


def _matmul_bias(x, w_t, b, row_blk):
    rows, k = x.shape
    cols = w_t.shape[1]
    grid = rows // row_blk
    return pl.pallas_call(
        _matmul_bias_body,
        grid=(grid,),
        in_specs=[
            pl.BlockSpec((row_blk, k), lambda i: (i, 0)),
            pl.BlockSpec((k, cols), lambda i: (0, 0)),
            pl.BlockSpec((1, cols), lambda i: (0, 0)),
        ],
        out_specs=pl.BlockSpec((row_blk, cols), lambda i: (i, 0)),
        out_shape=jax.ShapeDtypeStruct((rows, cols), jnp.float32),
    )(x, w_t, b)


def _combine_body(a_ref, b_ref, n_ref, o_ref):
    o_ref[...] = a_ref[0] + b_ref[0] - n_ref[...]


def _combine(parts, node):
    row_blk = 2000
    return pl.pallas_call(
        _combine_body,
        grid=(N // row_blk,),
        in_specs=[
            pl.BlockSpec((1, row_blk, H), lambda i: (0, i, 0)),
            pl.BlockSpec((1, row_blk, H), lambda i: (1, i, 0)),
            pl.BlockSpec((row_blk, H), lambda i: (i, 0)),
        ],
        out_specs=pl.BlockSpec((row_blk, H), lambda i: (i, 0)),
        out_shape=jax.ShapeDtypeStruct((N, H), jnp.float32),
    )(parts, parts, node)


NUM_GRAPHS_OUT = 64
MLP_BLK = 2000


def _mlp_pool_body(pa_ref, pb_ref, n_ref, w1_ref, b1_ref, w2_ref, b2_ref,
                   w3_ref, batch_ref, o_ref):
    i = pl.program_id(0)
    h = pa_ref[0] + pb_ref[0] - n_ref[...]
    h = jnp.maximum(h, 0.0)
    h = jnp.dot(h, w1_ref[...], preferred_element_type=jnp.float32) + b1_ref[...]
    h = jnp.maximum(h, 0.0)
    h = jnp.dot(h, w2_ref[...], preferred_element_type=jnp.float32) + b2_ref[...]
    h = jnp.maximum(h, 0.0)
    e = jnp.dot(h, w3_ref[...], preferred_element_type=jnp.float32)
    b = batch_ref[...].reshape(MLP_BLK)
    ids = lax.broadcasted_iota(jnp.int32, (MLP_BLK, NUM_GRAPHS_OUT), 1)
    oh = (b[:, None] == ids).astype(jnp.float32)
    dgp = lax.dot_general(oh, e, (((0,), (0,)), ((), ())),
                          preferred_element_type=jnp.float32)

    @pl.when(i == 0)
    def _():
        o_ref[...] = jnp.zeros_like(o_ref)

    o_ref[...] += dgp


def _mlp_pool(parts, node, w1_t, b1, w2_t, b2, w3_t, batch3):
    grid = N // MLP_BLK
    return pl.pallas_call(
        _mlp_pool_body,
        grid=(grid,),
        in_specs=[
            pl.BlockSpec((1, MLP_BLK, H), lambda i: (0, i, 0)),
            pl.BlockSpec((1, MLP_BLK, H), lambda i: (1, i, 0)),
            pl.BlockSpec((MLP_BLK, H), lambda i: (i, 0)),
            pl.BlockSpec((H, H), lambda i: (0, 0)),
            pl.BlockSpec((1, H), lambda i: (0, 0)),
            pl.BlockSpec((H, H // 2), lambda i: (0, 0)),
            pl.BlockSpec((1, H // 2), lambda i: (0, 0)),
            pl.BlockSpec((H // 2, 1), lambda i: (0, 0)),
            pl.BlockSpec((1, 1, MLP_BLK), lambda i: (i, 0, 0)),
        ],
        out_specs=pl.BlockSpec((NUM_GRAPHS_OUT, 1), lambda i: (0, 0)),
        out_shape=jax.ShapeDtypeStruct((NUM_GRAPHS_OUT, 1), jnp.float32),
    )(parts, parts, node, w1_t, b1, w2_t, b2, w3_t, batch3)


def kernel(x, edge_index, edge_attr, batch, W_atom, b_atom, W_bond, b_bond,
           W1, b1, W2, b2, W3):
    src = edge_index[0].astype(jnp.int32)
    dst = edge_index[1].astype(jnp.int32)
    batch3 = batch.astype(jnp.int32).reshape(N // MLP_BLK, 1, MLP_BLK)

    node_emb = _matmul_bias(x, W_atom.T, b_atom.reshape(1, H), 2000)
    edge_emb = _matmul_bias(edge_attr, W_bond.T, b_bond.reshape(1, H), 4000)

    parts1 = _sc_layer(node_emb, edge_emb, src, dst)
    node1 = _combine(parts1, node_emb)
    parts2 = _sc_layer(node1, edge_emb, src, dst)

    dg = _mlp_pool(parts2, node1, W1.T, b1.reshape(1, H),
                   W2.T, b2.reshape(1, H // 2), W3.T, batch3)
    return dg
